# baseline (speedup 1.0000x reference)
import functools

import jax
import jax.numpy as jnp
from jax import lax
from jax.experimental import pallas as pl
from jax.experimental.pallas import tpu as pltpu
from jax.experimental.pallas import tpu_sc as plsc

N_NODES = 50000
N_EDGES = 800000
D_IN = 100
DP = 128
R = 784
N_RANGES = 64
N_NODES_P = N_RANGES * R
C = 6400
N_CHUNKS = N_EDGES // C
G = 16
B_TC = 4000


def _mm_body(x_ref, w_ref, b_ref, o_ref):
    y = jnp.dot(x_ref[...], w_ref[...], preferred_element_type=jnp.float32)
    o_ref[...] = jnp.maximum(y + b_ref[...], 0.0)


def _tc_h(src_emb, w_pad, b_pad):
    return pl.pallas_call(
        _mm_body,
        grid=(N_EDGES // B_TC,),
        in_specs=[
            pl.BlockSpec((B_TC, D_IN), lambda i: (i, 0)),
            pl.BlockSpec((D_IN, DP), lambda i: (0, 0)),
            pl.BlockSpec((1, DP), lambda i: (0, 0)),
        ],
        out_specs=pl.BlockSpec((B_TC, DP), lambda i: (i, 0)),
        out_shape=jax.ShapeDtypeStruct((N_EDGES, DP), jnp.float32),
    )(src_emb, w_pad, b_pad)


def _splat_lane(v, r):
    idx = jnp.full((16,), r, jnp.int32)
    return lax.gather(
        v,
        idx[:, None],
        lax.GatherDimensionNumbers(
            offset_dims=(), collapsed_slice_dims=(0,), start_index_map=(0,)
        ),
        slice_sizes=(1,),
        mode=lax.GatherScatterMode.PROMISE_IN_BOUNDS,
    )


def _sc_body(dst_hbm, h_hbm, out_hbm, dstc, mids, rows0, rows1, acc, sem0, sem1):
    nc = 2
    wid = lax.axis_index("s") * nc + lax.axis_index("c")
    iota16 = lax.iota(jnp.int32, 16)
    neg_inf = jnp.full((16,), -jnp.inf, jnp.float32)

    def fire(g, rows_ref, sem):
        ids = jnp.clip(mids[pl.ds(g * G, G)], 0, N_EDGES - 1)
        pltpu.make_async_copy(h_hbm.at[ids], rows_ref, sem).start()

    def drain(rows_ref, sem):
        pltpu.make_async_copy(h_hbm.at[pl.ds(0, G)], rows_ref, sem).wait()

    for p in range(2):
        rid = wid * 2 + p
        lo = rid * R

        def init_body(i, _):
            acc[pl.ds(i * 16, 16)] = neg_inf
            return 0

        lax.fori_loop(0, (R + 1) * DP // 16, init_body, 0)

        def chunk_body(c, _):
            cbase = c * C
            pltpu.sync_copy(dst_hbm.at[pl.ds(cbase, C)], dstc)

            def fbody(i, cnt_splat):
                v = dstc[pl.ds(i * 16, 16)]
                m = (v >= lo) & (v < lo + R)
                eid = lax.iota(jnp.int32, 16) + (cbase + i * 16)
                _, sv = plsc.sort_key_val(
                    m.astype(jnp.int32), eid, descending=True
                )
                plsc.store_scatter(
                    mids, [cnt_splat + lax.iota(jnp.int32, 16)], sv
                )
                return cnt_splat + plsc.all_reduce_population_count(m)

            cnt_splat = lax.fori_loop(
                0, C // 16, fbody, jnp.zeros((16,), jnp.int32)
            )
            cnt = cnt_splat[0]
            ng = (cnt + (G - 1)) // G

            def process_group(g, rows_ref):
                ids = jnp.clip(mids[pl.ds(g * G, G)], 0, N_EDGES - 1)
                real = (g * G + iota16) < cnt_splat
                dv = plsc.load_gather(dstc, [jnp.clip(ids - cbase, 0, C - 1)])
                d = jnp.where(real, dv - lo, R)
                for r in range(G):
                    base = _splat_lane(d, r) * DP
                    for k in range(DP // 16):
                        idx = base + (16 * k) + iota16
                        upd = rows_ref[r, 16 * k:16 * (k + 1)]
                        cur = plsc.load_gather(acc, [idx])
                        plsc.store_scatter(acc, [idx], jnp.maximum(cur, upd))

            @pl.when(ng > 0)
            def _():
                fire(0, rows0, sem0)

                def gbody(g, _):
                    @pl.when(g + 1 < ng)
                    def _():
                        @pl.when(lax.rem(g, 2) == 1)
                        def _():
                            fire(g + 1, rows0, sem0)

                        @pl.when(lax.rem(g, 2) == 0)
                        def _():
                            fire(g + 1, rows1, sem1)

                    @pl.when(lax.rem(g, 2) == 0)
                    def _():
                        drain(rows0, sem0)
                        process_group(g, rows0)

                    @pl.when(lax.rem(g, 2) == 1)
                    def _():
                        drain(rows1, sem1)
                        process_group(g, rows1)

                    return 0

                lax.fori_loop(0, ng, gbody, 0)

            return 0

        lax.fori_loop(0, N_CHUNKS, chunk_body, 0)

        pltpu.sync_copy(
            acc.at[pl.ds(0, R * DP)], out_hbm.at[pl.ds(rid * R * DP, R * DP)]
        )


def _sc_scatter_max(dst, h):
    mesh = plsc.VectorSubcoreMesh(core_axis_name="c", subcore_axis_name="s")
    f = functools.partial(
        pl.kernel,
        mesh=mesh,
        out_type=jax.ShapeDtypeStruct((N_NODES_P * DP,), jnp.float32),
        scratch_types=[
            pltpu.VMEM((C,), jnp.int32),
            pltpu.VMEM((C + G,), jnp.int32),
            pltpu.VMEM((G, DP), jnp.float32),
            pltpu.VMEM((G, DP), jnp.float32),
            pltpu.VMEM(((R + 1) * DP,), jnp.float32),
            pltpu.SemaphoreType.DMA,
            pltpu.SemaphoreType.DMA,
        ],
        compiler_params=pltpu.CompilerParams(needs_layout_passes=False),
    )(_sc_body)
    return f(dst, h)


@jax.jit
def kernel(edge_index, src_emb, src_emb_in, W, b):
    del src_emb_in
    w_pad = jnp.zeros((D_IN, DP), jnp.float32).at[:, :D_IN].set(W.T)
    b_pad = jnp.zeros((1, DP), jnp.float32).at[0, :D_IN].set(b)
    h = _tc_h(src_emb, w_pad, b_pad)
    dst = edge_index[1]
    out = _sc_scatter_max(dst, h)
    return out.reshape(N_NODES_P, DP)[:N_NODES, :D_IN]


# speedup vs baseline: 1.2197x; 1.2197x over previous
import functools

import jax
import jax.numpy as jnp
from jax import lax
from jax.experimental import pallas as pl
from jax.experimental.pallas import tpu as pltpu
from jax.experimental.pallas import tpu_sc as plsc

N_NODES = 50000
N_EDGES = 800000
D_IN = 100
DP = 128
AS = 112
R = 784
N_RANGES = 64
N_NODES_P = N_RANGES * R
C = 4000
N_CHUNKS = N_EDGES // C
CAP = 4096
SG = 64
B_TC = 4000


def _mm_body(x_ref, w_ref, b_ref, o_ref):
    y = jnp.dot(x_ref[...], w_ref[...], preferred_element_type=jnp.float32)
    o_ref[...] = jnp.maximum(y + b_ref[...], 0.0)


def _tc_h(src_emb, w_pad, b_pad):
    return pl.pallas_call(
        _mm_body,
        grid=(N_EDGES // B_TC,),
        in_specs=[
            pl.BlockSpec((B_TC, D_IN), lambda i: (i, 0)),
            pl.BlockSpec((D_IN, DP), lambda i: (0, 0)),
            pl.BlockSpec((1, DP), lambda i: (0, 0)),
        ],
        out_specs=pl.BlockSpec((B_TC, DP), lambda i: (i, 0)),
        out_shape=jax.ShapeDtypeStruct((N_EDGES, DP), jnp.float32),
    )(src_emb, w_pad, b_pad)


def _splat_lane(v, r):
    idx = jnp.full((16,), r, jnp.int32)
    return lax.gather(
        v,
        idx[:, None],
        lax.GatherDimensionNumbers(
            offset_dims=(), collapsed_slice_dims=(0,), start_index_map=(0,)
        ),
        slice_sizes=(1,),
        mode=lax.GatherScatterMode.PROMISE_IN_BOUNDS,
    )


def _sc_body(dst_hbm, h_hbm, out_hbm, dstc, queue, rows0, rows1, idx0, idx1,
             acc, sem0, sem1, csem0, csem1):
    nc = 2
    wid = lax.axis_index("s") * nc + lax.axis_index("c")
    iota16 = lax.iota(jnp.int32, 16)
    neg_inf = jnp.full((16,), -jnp.inf, jnp.float32)

    def chunk_fire(c, half, csem):
        pltpu.make_async_copy(
            dst_hbm.at[pl.ds(c * C, C)], dstc.at[pl.ds(half * C, C)], csem
        ).start()

    def chunk_drain(half, csem):
        pltpu.make_async_copy(
            dst_hbm.at[pl.ds(0, C)], dstc.at[pl.ds(half * C, C)], csem
        ).wait()

    def stage_and_fire(qblk, idx_ref, rows_ref, sem):
        qoff = (qblk * SG) & (CAP - 1)
        for sub in range(SG // 16):
            qv = queue[pl.ds(qoff + sub * 16, 16)]
            ids = jnp.minimum(qv & 0xFFFFF, N_EDGES - 1)
            idx_ref[sub * 16:(sub + 1) * 16] = ids
        pltpu.make_async_copy(h_hbm.at[idx_ref], rows_ref, sem).start()

    def drain_rows(rows_ref, sem):
        pltpu.make_async_copy(h_hbm.at[pl.ds(0, SG)], rows_ref, sem).wait()

    def process_block(qblk, cnt_splat, rows_ref):
        qoff = (qblk * SG) & (CAP - 1)
        for sub in range(SG // 16):
            qv = queue[pl.ds(qoff + sub * 16, 16)]
            d_raw = lax.shift_right_logical(qv, 20)
            real = (qblk * SG + sub * 16 + iota16) < cnt_splat
            d = jnp.where(real, d_raw, R)
            for r in range(16):
                base = _splat_lane(d, r) * AS
                row = sub * 16 + r
                curs = []
                upds = []
                for k in range(AS // 16):
                    idx = base + (16 * k) + iota16
                    curs.append((idx, plsc.load_gather(acc, [idx])))
                    upds.append(rows_ref[row, 16 * k:16 * (k + 1)])
                for k in range(AS // 16):
                    idx, cur = curs[k]
                    plsc.store_scatter(acc, [idx], jnp.maximum(cur, upds[k]))

    def pass_body(p, _):
        rid = wid * 2 + p
        lo = rid * R

        def init_body(i, _):
            acc[pl.ds(i * 16, 16)] = neg_inf
            return 0

        lax.fori_loop(0, (R + 1) * AS // 16, init_body, 0)

        chunk_fire(0, 0, csem0)

        def chunk_body(c, carry):
            cnt_splat, nqb = carry
            half = lax.rem(c, 2)
            cbase = c * C

            @pl.when(half == 0)
            def _():
                chunk_drain(0, csem0)

            @pl.when(half == 1)
            def _():
                chunk_drain(1, csem1)

            @pl.when(c + 1 < N_CHUNKS)
            def _():
                @pl.when(half == 0)
                def _():
                    chunk_fire(c + 1, 1, csem1)

                @pl.when(half == 1)
                def _():
                    chunk_fire(c + 1, 0, csem0)

            dbase = half * C

            def fbody(i, cs):
                v = dstc[pl.ds(dbase + i * 16, 16)]
                m = (v >= lo) & (v < lo + R)
                packed = (lax.iota(jnp.int32, 16) + (cbase + i * 16)) | (
                    (v - lo) << 20
                )
                _, sv = plsc.sort_key_val(
                    m.astype(jnp.int32), packed, descending=True
                )
                pop = plsc.all_reduce_population_count(m)
                smask = iota16 < pop
                pos = (cs + iota16) & (CAP - 1)
                plsc.store_scatter(queue, [pos], sv, mask=smask)
                return cs + pop

            cnt_splat = lax.fori_loop(0, C // 16, fbody, cnt_splat)
            cnt = cnt_splat[0]
            extra = jnp.where(c == N_CHUNKS - 1, SG - 1, 0)
            nblk = (cnt - nqb * SG + extra) // SG

            @pl.when(nblk > 0)
            def _():
                @pl.when(lax.rem(nqb, 2) == 0)
                def _():
                    stage_and_fire(nqb, idx0, rows0, sem0)

                @pl.when(lax.rem(nqb, 2) == 1)
                def _():
                    stage_and_fire(nqb, idx1, rows1, sem1)

                def bbody(b, _):
                    qb = nqb + b

                    @pl.when(b + 1 < nblk)
                    def _():
                        @pl.when(lax.rem(qb + 1, 2) == 0)
                        def _():
                            stage_and_fire(qb + 1, idx0, rows0, sem0)

                        @pl.when(lax.rem(qb + 1, 2) == 1)
                        def _():
                            stage_and_fire(qb + 1, idx1, rows1, sem1)

                    @pl.when(lax.rem(qb, 2) == 0)
                    def _():
                        drain_rows(rows0, sem0)
                        process_block(qb, cnt_splat, rows0)

                    @pl.when(lax.rem(qb, 2) == 1)
                    def _():
                        drain_rows(rows1, sem1)
                        process_block(qb, cnt_splat, rows1)

                    return 0

                lax.fori_loop(0, nblk, bbody, 0)

            return (cnt_splat, nqb + nblk)

        lax.fori_loop(
            0, N_CHUNKS, chunk_body, (jnp.zeros((16,), jnp.int32), jnp.int32(0))
        )

        pltpu.sync_copy(
            acc.at[pl.ds(0, R * AS)], out_hbm.at[pl.ds(rid * R * AS, R * AS)]
        )
        return 0

    lax.fori_loop(0, 2, pass_body, 0)


def _sc_scatter_max(dst, h):
    mesh = plsc.VectorSubcoreMesh(core_axis_name="c", subcore_axis_name="s")
    f = functools.partial(
        pl.kernel,
        mesh=mesh,
        out_type=jax.ShapeDtypeStruct((N_NODES_P * AS,), jnp.float32),
        scratch_types=[
            pltpu.VMEM((2 * C,), jnp.int32),
            pltpu.VMEM((CAP,), jnp.int32),
            pltpu.VMEM((SG, DP), jnp.float32),
            pltpu.VMEM((SG, DP), jnp.float32),
            pltpu.VMEM((SG,), jnp.int32),
            pltpu.VMEM((SG,), jnp.int32),
            pltpu.VMEM(((R + 1) * AS,), jnp.float32),
            pltpu.SemaphoreType.DMA,
            pltpu.SemaphoreType.DMA,
            pltpu.SemaphoreType.DMA,
            pltpu.SemaphoreType.DMA,
        ],
        compiler_params=pltpu.CompilerParams(needs_layout_passes=False),
    )(_sc_body)
    return f(dst, h)


@jax.jit
def kernel(edge_index, src_emb, src_emb_in, W, b):
    del src_emb_in
    w_pad = jnp.zeros((D_IN, DP), jnp.float32).at[:, :D_IN].set(W.T)
    b_pad = jnp.zeros((1, DP), jnp.float32).at[0, :D_IN].set(b)
    h = _tc_h(src_emb, w_pad, b_pad)
    dst = edge_index[1]
    out = _sc_scatter_max(dst, h)
    return out.reshape(N_NODES_P, AS)[:N_NODES, :D_IN]


# speedup vs baseline: 1.5032x; 1.2324x over previous
import functools

import jax
import jax.numpy as jnp
from jax import lax
from jax.experimental import pallas as pl
from jax.experimental.pallas import tpu as pltpu
from jax.experimental.pallas import tpu_sc as plsc

N_NODES = 50000
N_EDGES = 800000
D_IN = 100
DP = 128
AS = 112
R = 784
N_RANGES = 64
N_NODES_P = N_RANGES * R
C = 4000
N_CHUNKS = N_EDGES // C
CAP = 4096
SG = 64
FU = 5
B_TC = 4000


def _mm_body(x_ref, w_ref, b_ref, o_ref):
    y = jnp.dot(x_ref[...], w_ref[...], preferred_element_type=jnp.float32)
    o_ref[...] = jnp.maximum(y + b_ref[...], 0.0)


def _tc_h(src_emb, w_pad, b_pad):
    return pl.pallas_call(
        _mm_body,
        grid=(N_EDGES // B_TC,),
        in_specs=[
            pl.BlockSpec((B_TC, D_IN), lambda i: (i, 0)),
            pl.BlockSpec((D_IN, DP), lambda i: (0, 0)),
            pl.BlockSpec((1, DP), lambda i: (0, 0)),
        ],
        out_specs=pl.BlockSpec((B_TC, DP), lambda i: (i, 0)),
        out_shape=jax.ShapeDtypeStruct((N_EDGES, DP), jnp.float32),
    )(src_emb, w_pad, b_pad)


def _splat_lane(v, r):
    idx = jnp.full((16,), r, jnp.int32)
    return lax.gather(
        v,
        idx[:, None],
        lax.GatherDimensionNumbers(
            offset_dims=(), collapsed_slice_dims=(0,), start_index_map=(0,)
        ),
        slice_sizes=(1,),
        mode=lax.GatherScatterMode.PROMISE_IN_BOUNDS,
    )


def _sc_body(dst_hbm, h_hbm, out_hbm, dstc, queue, rows0, rows1, idx0, idx1,
             acc, sem0, sem1, csem0, csem1):
    nc = 2
    wid = lax.axis_index("s") * nc + lax.axis_index("c")
    iota16 = lax.iota(jnp.int32, 16)
    neg_inf = jnp.full((16,), -jnp.inf, jnp.float32)

    def chunk_fire(c, half, csem):
        pltpu.make_async_copy(
            dst_hbm.at[pl.ds(c * C, C)], dstc.at[pl.ds(half * C, C)], csem
        ).start()

    def chunk_drain(half, csem):
        pltpu.make_async_copy(
            dst_hbm.at[pl.ds(0, C)], dstc.at[pl.ds(half * C, C)], csem
        ).wait()

    def stage_and_fire(qblk, idx_ref, rows_ref, sem):
        qoff = (qblk * SG) & (CAP - 1)
        for sub in range(SG // 16):
            qv = queue[pl.ds(qoff + sub * 16, 16)]
            ids = jnp.minimum(qv & 0xFFFFF, N_EDGES - 1)
            idx_ref[sub * 16:(sub + 1) * 16] = ids
        pltpu.make_async_copy(h_hbm.at[idx_ref], rows_ref, sem).start()

    def drain_rows(rows_ref, sem):
        pltpu.make_async_copy(h_hbm.at[pl.ds(0, SG)], rows_ref, sem).wait()

    def process_block(qblk, cnt_splat, rows_ref):
        qoff = (qblk * SG) & (CAP - 1)
        for sub in range(SG // 16):
            qv = queue[pl.ds(qoff + sub * 16, 16)]
            d_raw = lax.shift_right_logical(qv, 20)
            real = (qblk * SG + sub * 16 + iota16) < cnt_splat
            d = jnp.where(real, d_raw, R)
            for r in range(16):
                base = _splat_lane(d, r) * AS
                row = sub * 16 + r
                curs = []
                upds = []
                for k in range(AS // 16):
                    idx = base + (16 * k) + iota16
                    curs.append((idx, plsc.load_gather(acc, [idx])))
                    upds.append(rows_ref[row, 16 * k:16 * (k + 1)])
                for k in range(AS // 16):
                    idx, cur = curs[k]
                    plsc.store_scatter(acc, [idx], jnp.maximum(cur, upds[k]))

    def pass_body(p, _):
        rid = wid * 2 + p
        lo = rid * R

        def init_body(i, _):
            acc[pl.ds(i * 16, 16)] = neg_inf
            return 0

        lax.fori_loop(0, (R + 1) * AS // 16, init_body, 0)

        chunk_fire(0, 0, csem0)

        def chunk_body(c, carry):
            cnt_splat, nqb = carry
            half = lax.rem(c, 2)
            cbase = c * C

            @pl.when(half == 0)
            def _():
                chunk_drain(0, csem0)

            @pl.when(half == 1)
            def _():
                chunk_drain(1, csem1)

            @pl.when(c + 1 < N_CHUNKS)
            def _():
                @pl.when(half == 0)
                def _():
                    chunk_fire(c + 1, 1, csem1)

                @pl.when(half == 1)
                def _():
                    chunk_fire(c + 1, 0, csem0)

            dbase = half * C

            def fbody(i, cs):
                pops = []
                svs = []
                for u in range(FU):
                    off = i * 16 * FU + u * 16
                    v = dstc[pl.ds(dbase + off, 16)]
                    m = (v >= lo) & (v < lo + R)
                    packed = (lax.iota(jnp.int32, 16) + (cbase + off)) | (
                        (v - lo) << 20
                    )
                    _, sv = plsc.sort_key_val(
                        m.astype(jnp.int32), packed, descending=True
                    )
                    svs.append(sv)
                    pops.append(plsc.all_reduce_population_count(m))
                csu = cs
                for u in range(FU):
                    smask = iota16 < pops[u]
                    pos = (csu + iota16) & (CAP - 1)
                    plsc.store_scatter(queue, [pos], svs[u], mask=smask)
                    csu = csu + pops[u]
                return csu

            cnt_splat = lax.fori_loop(0, C // (16 * FU), fbody, cnt_splat)
            cnt = cnt_splat[0]
            extra = jnp.where(c == N_CHUNKS - 1, SG - 1, 0)
            nblk = (cnt - nqb * SG + extra) // SG

            @pl.when(nblk > 0)
            def _():
                @pl.when(lax.rem(nqb, 2) == 0)
                def _():
                    stage_and_fire(nqb, idx0, rows0, sem0)

                @pl.when(lax.rem(nqb, 2) == 1)
                def _():
                    stage_and_fire(nqb, idx1, rows1, sem1)

                def bbody(b, _):
                    qb = nqb + b

                    @pl.when(b + 1 < nblk)
                    def _():
                        @pl.when(lax.rem(qb + 1, 2) == 0)
                        def _():
                            stage_and_fire(qb + 1, idx0, rows0, sem0)

                        @pl.when(lax.rem(qb + 1, 2) == 1)
                        def _():
                            stage_and_fire(qb + 1, idx1, rows1, sem1)

                    @pl.when(lax.rem(qb, 2) == 0)
                    def _():
                        drain_rows(rows0, sem0)
                        process_block(qb, cnt_splat, rows0)

                    @pl.when(lax.rem(qb, 2) == 1)
                    def _():
                        drain_rows(rows1, sem1)
                        process_block(qb, cnt_splat, rows1)

                    return 0

                lax.fori_loop(0, nblk, bbody, 0)

            return (cnt_splat, nqb + nblk)

        lax.fori_loop(
            0, N_CHUNKS, chunk_body, (jnp.zeros((16,), jnp.int32), jnp.int32(0))
        )

        pltpu.sync_copy(
            acc.at[pl.ds(0, R * AS)], out_hbm.at[pl.ds(rid * R * AS, R * AS)]
        )
        return 0

    lax.fori_loop(0, 2, pass_body, 0)


def _sc_scatter_max(dst, h):
    mesh = plsc.VectorSubcoreMesh(core_axis_name="c", subcore_axis_name="s")
    f = functools.partial(
        pl.kernel,
        mesh=mesh,
        out_type=jax.ShapeDtypeStruct((N_NODES_P * AS,), jnp.float32),
        scratch_types=[
            pltpu.VMEM((2 * C,), jnp.int32),
            pltpu.VMEM((CAP,), jnp.int32),
            pltpu.VMEM((SG, DP), jnp.float32),
            pltpu.VMEM((SG, DP), jnp.float32),
            pltpu.VMEM((SG,), jnp.int32),
            pltpu.VMEM((SG,), jnp.int32),
            pltpu.VMEM(((R + 1) * AS,), jnp.float32),
            pltpu.SemaphoreType.DMA,
            pltpu.SemaphoreType.DMA,
            pltpu.SemaphoreType.DMA,
            pltpu.SemaphoreType.DMA,
        ],
        compiler_params=pltpu.CompilerParams(needs_layout_passes=False),
    )(_sc_body)
    return f(dst, h)


@jax.jit
def kernel(edge_index, src_emb, src_emb_in, W, b):
    del src_emb_in
    w_pad = jnp.zeros((D_IN, DP), jnp.float32).at[:, :D_IN].set(W.T)
    b_pad = jnp.zeros((1, DP), jnp.float32).at[0, :D_IN].set(b)
    h = _tc_h(src_emb, w_pad, b_pad)
    dst = edge_index[1]
    out = _sc_scatter_max(dst, h)
    return out.reshape(N_NODES_P, AS)[:N_NODES, :D_IN]


# speedup vs baseline: 1.6397x; 1.0908x over previous
import functools

import jax
import jax.numpy as jnp
from jax import lax
from jax.experimental import pallas as pl
from jax.experimental.pallas import tpu as pltpu
from jax.experimental.pallas import tpu_sc as plsc

N_NODES = 50000
N_EDGES = 800000
D_IN = 100
DP = 128
AS = 112
R = 784
N_RANGES = 64
N_NODES_P = N_RANGES * R
C = 4000
N_CHUNKS = N_EDGES // C
CAP = 4096
SG = 64
FU = 10
B_TC = 4000


def _mm_body(x_ref, w_ref, b_ref, o_ref):
    y = jnp.dot(x_ref[...], w_ref[...], preferred_element_type=jnp.float32)
    o_ref[...] = jnp.maximum(y + b_ref[...], 0.0)


def _tc_h(src_emb, w_pad, b_pad):
    return pl.pallas_call(
        _mm_body,
        grid=(N_EDGES // B_TC,),
        in_specs=[
            pl.BlockSpec((B_TC, D_IN), lambda i: (i, 0)),
            pl.BlockSpec((D_IN, DP), lambda i: (0, 0)),
            pl.BlockSpec((1, DP), lambda i: (0, 0)),
        ],
        out_specs=pl.BlockSpec((B_TC, DP), lambda i: (i, 0)),
        out_shape=jax.ShapeDtypeStruct((N_EDGES, DP), jnp.float32),
    )(src_emb, w_pad, b_pad)


def _splat_lane(v, r):
    idx = jnp.full((16,), r, jnp.int32)
    return lax.gather(
        v,
        idx[:, None],
        lax.GatherDimensionNumbers(
            offset_dims=(), collapsed_slice_dims=(0,), start_index_map=(0,)
        ),
        slice_sizes=(1,),
        mode=lax.GatherScatterMode.PROMISE_IN_BOUNDS,
    )


def _sc_body(dst_hbm, h_hbm, out_hbm, dstc, queue, rows0, rows1, idx0, idx1,
             acc, sem0, sem1, csem0, csem1):
    nc = 2
    wid = lax.axis_index("s") * nc + lax.axis_index("c")
    iota16 = lax.iota(jnp.int32, 16)
    neg_inf = jnp.full((16,), -jnp.inf, jnp.float32)

    def chunk_fire(c, half, csem):
        pltpu.make_async_copy(
            dst_hbm.at[pl.ds(c * C, C)], dstc.at[pl.ds(half * C, C)], csem
        ).start()

    def chunk_drain(half, csem):
        pltpu.make_async_copy(
            dst_hbm.at[pl.ds(0, C)], dstc.at[pl.ds(half * C, C)], csem
        ).wait()

    def stage_and_fire(qblk, idx_ref, rows_ref, sem):
        qoff = (qblk * SG) & (CAP - 1)
        for sub in range(SG // 16):
            qv = queue[pl.ds(qoff + sub * 16, 16)]
            ids = jnp.minimum(qv & 0xFFFFF, N_EDGES - 1)
            idx_ref[sub * 16:(sub + 1) * 16] = ids
        pltpu.make_async_copy(h_hbm.at[idx_ref], rows_ref, sem).start()

    def drain_rows(rows_ref, sem):
        pltpu.make_async_copy(h_hbm.at[pl.ds(0, SG)], rows_ref, sem).wait()

    def process_block(qblk, cnt_splat, rows_ref):
        qoff = (qblk * SG) & (CAP - 1)
        for sub in range(SG // 16):
            qv = queue[pl.ds(qoff + sub * 16, 16)]
            d_raw = lax.shift_right_logical(qv, 20)
            real = (qblk * SG + sub * 16 + iota16) < cnt_splat
            d = jnp.where(real, d_raw, R)
            for r in range(16):
                base = _splat_lane(d, r) * AS
                row = sub * 16 + r
                curs = []
                upds = []
                for k in range(AS // 16):
                    idx = base + (16 * k) + iota16
                    curs.append((idx, plsc.load_gather(acc, [idx])))
                    upds.append(rows_ref[row, 16 * k:16 * (k + 1)])
                for k in range(AS // 16):
                    idx, cur = curs[k]
                    plsc.store_scatter(acc, [idx], jnp.maximum(cur, upds[k]))

    def pass_body(p, _):
        rid = wid * 2 + p
        lo = rid * R

        def init_body(i, _):
            acc[pl.ds(i * 16, 16)] = neg_inf
            return 0

        lax.fori_loop(0, (R + 1) * AS // 16, init_body, 0)

        chunk_fire(0, 0, csem0)

        def chunk_body(c, carry):
            cnt_splat, nqb = carry
            half = lax.rem(c, 2)
            cbase = c * C

            @pl.when(half == 0)
            def _():
                chunk_drain(0, csem0)

            @pl.when(half == 1)
            def _():
                chunk_drain(1, csem1)

            @pl.when(c + 1 < N_CHUNKS)
            def _():
                @pl.when(half == 0)
                def _():
                    chunk_fire(c + 1, 1, csem1)

                @pl.when(half == 1)
                def _():
                    chunk_fire(c + 1, 0, csem0)

            dbase = half * C

            def fbody(i, cs):
                pops = []
                svs = []
                for u in range(FU):
                    off = i * 16 * FU + u * 16
                    v = dstc[pl.ds(dbase + off, 16)]
                    m = (v >= lo) & (v < lo + R)
                    packed = (lax.iota(jnp.int32, 16) + (cbase + off)) | (
                        (v - lo) << 20
                    )
                    _, sv = plsc.sort_key_val(
                        m.astype(jnp.int32), packed, descending=True
                    )
                    svs.append(sv)
                    pops.append(plsc.all_reduce_population_count(m))
                csu = cs
                for u in range(FU):
                    smask = iota16 < pops[u]
                    pos = (csu + iota16) & (CAP - 1)
                    plsc.store_scatter(queue, [pos], svs[u], mask=smask)
                    csu = csu + pops[u]
                return csu

            cnt_splat = lax.fori_loop(0, C // (16 * FU), fbody, cnt_splat)
            cnt = cnt_splat[0]
            extra = jnp.where(c == N_CHUNKS - 1, SG - 1, 0)
            nblk = (cnt - nqb * SG + extra) // SG

            @pl.when(nblk > 0)
            def _():
                @pl.when(lax.rem(nqb, 2) == 0)
                def _():
                    stage_and_fire(nqb, idx0, rows0, sem0)

                @pl.when(lax.rem(nqb, 2) == 1)
                def _():
                    stage_and_fire(nqb, idx1, rows1, sem1)

                def bbody(b, _):
                    qb = nqb + b

                    @pl.when(b + 1 < nblk)
                    def _():
                        @pl.when(lax.rem(qb + 1, 2) == 0)
                        def _():
                            stage_and_fire(qb + 1, idx0, rows0, sem0)

                        @pl.when(lax.rem(qb + 1, 2) == 1)
                        def _():
                            stage_and_fire(qb + 1, idx1, rows1, sem1)

                    @pl.when(lax.rem(qb, 2) == 0)
                    def _():
                        drain_rows(rows0, sem0)
                        process_block(qb, cnt_splat, rows0)

                    @pl.when(lax.rem(qb, 2) == 1)
                    def _():
                        drain_rows(rows1, sem1)
                        process_block(qb, cnt_splat, rows1)

                    return 0

                lax.fori_loop(0, nblk, bbody, 0)

            return (cnt_splat, nqb + nblk)

        lax.fori_loop(
            0, N_CHUNKS, chunk_body, (jnp.zeros((16,), jnp.int32), jnp.int32(0))
        )

        pltpu.sync_copy(
            acc.at[pl.ds(0, R * AS)], out_hbm.at[pl.ds(rid * R * AS, R * AS)]
        )
        return 0

    lax.fori_loop(0, 2, pass_body, 0)


def _sc_scatter_max(dst, h):
    mesh = plsc.VectorSubcoreMesh(core_axis_name="c", subcore_axis_name="s")
    f = functools.partial(
        pl.kernel,
        mesh=mesh,
        out_type=jax.ShapeDtypeStruct((N_NODES_P * AS,), jnp.float32),
        scratch_types=[
            pltpu.VMEM((2 * C,), jnp.int32),
            pltpu.VMEM((CAP,), jnp.int32),
            pltpu.VMEM((SG, DP), jnp.float32),
            pltpu.VMEM((SG, DP), jnp.float32),
            pltpu.VMEM((SG,), jnp.int32),
            pltpu.VMEM((SG,), jnp.int32),
            pltpu.VMEM(((R + 1) * AS,), jnp.float32),
            pltpu.SemaphoreType.DMA,
            pltpu.SemaphoreType.DMA,
            pltpu.SemaphoreType.DMA,
            pltpu.SemaphoreType.DMA,
        ],
        compiler_params=pltpu.CompilerParams(needs_layout_passes=False),
    )(_sc_body)
    return f(dst, h)


@jax.jit
def kernel(edge_index, src_emb, src_emb_in, W, b):
    del src_emb_in
    w_pad = jnp.zeros((D_IN, DP), jnp.float32).at[:, :D_IN].set(W.T)
    b_pad = jnp.zeros((1, DP), jnp.float32).at[0, :D_IN].set(b)
    h = _tc_h(src_emb, w_pad, b_pad)
    dst = edge_index[1]
    out = _sc_scatter_max(dst, h)
    return out.reshape(N_NODES_P, AS)[:N_NODES, :D_IN]


# speedup vs baseline: 1.8427x; 1.1238x over previous
import functools

import jax
import jax.numpy as jnp
from jax import lax
from jax.experimental import pallas as pl
from jax.experimental.pallas import tpu as pltpu
from jax.experimental.pallas import tpu_sc as plsc

N_NODES = 50000
N_EDGES = 800000
D_IN = 100
DP = 128
AS = 112
R = 784
N_RANGES = 64
N_NODES_P = N_RANGES * R
C = 4000
N_CHUNKS = N_EDGES // C
CAP = 8192
SG = 64
FU = 10
B_TC = 4000


def _mm_body(x_ref, w_ref, b_ref, o_ref):
    y = jnp.dot(x_ref[...], w_ref[...], preferred_element_type=jnp.float32)
    o_ref[...] = jnp.maximum(y + b_ref[...], 0.0)


def _tc_h(src_emb, w_pad, b_pad):
    return pl.pallas_call(
        _mm_body,
        grid=(N_EDGES // B_TC,),
        in_specs=[
            pl.BlockSpec((B_TC, D_IN), lambda i: (i, 0)),
            pl.BlockSpec((D_IN, DP), lambda i: (0, 0)),
            pl.BlockSpec((1, DP), lambda i: (0, 0)),
        ],
        out_specs=pl.BlockSpec((B_TC, DP), lambda i: (i, 0)),
        out_shape=jax.ShapeDtypeStruct((N_EDGES, DP), jnp.float32),
    )(src_emb, w_pad, b_pad)


def _splat_lane(v, r):
    idx = jnp.full((16,), r, jnp.int32)
    return lax.gather(
        v,
        idx[:, None],
        lax.GatherDimensionNumbers(
            offset_dims=(), collapsed_slice_dims=(0,), start_index_map=(0,)
        ),
        slice_sizes=(1,),
        mode=lax.GatherScatterMode.PROMISE_IN_BOUNDS,
    )


def _sc_body(dst_hbm, h_hbm, out_hbm, dstc, queue, rows0, rows1, idx0, idx1,
             acc, sem0, sem1, csem0, csem1):
    nc = 2
    wid = lax.axis_index("s") * nc + lax.axis_index("c")
    iota16 = lax.iota(jnp.int32, 16)
    neg_inf = jnp.full((16,), -jnp.inf, jnp.float32)

    def chunk_fire(c, half, csem):
        pltpu.make_async_copy(
            dst_hbm.at[pl.ds(c * C, C)], dstc.at[pl.ds(half * C, C)], csem
        ).start()

    def chunk_drain(half, csem):
        pltpu.make_async_copy(
            dst_hbm.at[pl.ds(0, C)], dstc.at[pl.ds(half * C, C)], csem
        ).wait()

    def stage_and_fire(qblk, idx_ref, rows_ref, sem):
        qoff = (qblk * SG) & (CAP - 1)
        for sub in range(SG // 16):
            qv = queue[pl.ds(qoff + sub * 16, 16)]
            ids = jnp.minimum(qv & 0xFFFFF, N_EDGES - 1)
            idx_ref[sub * 16:(sub + 1) * 16] = ids
        pltpu.make_async_copy(h_hbm.at[idx_ref], rows_ref, sem).start()

    def drain_rows(rows_ref, sem):
        pltpu.make_async_copy(h_hbm.at[pl.ds(0, SG)], rows_ref, sem).wait()

    def process_block(qblk, cnt_splat, rows_ref):
        qoff = (qblk * SG) & (CAP - 1)
        for sub in range(SG // 16):
            qv = queue[pl.ds(qoff + sub * 16, 16)]
            d_raw = lax.shift_right_logical(qv, 20)
            real = (qblk * SG + sub * 16 + iota16) < cnt_splat
            d = jnp.where(real, d_raw, R)
            for r in range(16):
                base = _splat_lane(d, r) * AS
                row = sub * 16 + r
                curs = []
                upds = []
                for k in range(AS // 16):
                    idx = base + (16 * k) + iota16
                    curs.append((idx, plsc.load_gather(acc, [idx])))
                    upds.append(rows_ref[row, 16 * k:16 * (k + 1)])
                for k in range(AS // 16):
                    idx, cur = curs[k]
                    plsc.store_scatter(acc, [idx], jnp.maximum(cur, upds[k]))

    def pass_body(p, _):
        rid = wid * 2 + p
        lo = rid * R

        def init_body(i, _):
            acc[pl.ds(i * 16, 16)] = neg_inf
            return 0

        lax.fori_loop(0, (R + 1) * AS // 16, init_body, 0)

        chunk_fire(0, 0, csem0)

        def chunk_body(c, carry):
            cnt_splat, nfired = carry
            half = lax.rem(c, 2)
            cbase = c * C

            @pl.when(half == 0)
            def _():
                chunk_drain(0, csem0)

            @pl.when(half == 1)
            def _():
                chunk_drain(1, csem1)

            @pl.when(c + 1 < N_CHUNKS)
            def _():
                @pl.when(half == 0)
                def _():
                    chunk_fire(c + 1, 1, csem1)

                @pl.when(half == 1)
                def _():
                    chunk_fire(c + 1, 0, csem0)

            dbase = half * C

            def fbody(i, cs):
                pops = []
                svs = []
                for u in range(FU):
                    off = i * 16 * FU + u * 16
                    v = dstc[pl.ds(dbase + off, 16)]
                    m = (v >= lo) & (v < lo + R)
                    packed = (lax.iota(jnp.int32, 16) + (cbase + off)) | (
                        (v - lo) << 20
                    )
                    _, sv = plsc.sort_key_val(
                        m.astype(jnp.int32), packed, descending=True
                    )
                    svs.append(sv)
                    pops.append(plsc.all_reduce_population_count(m))
                csu = cs
                for u in range(FU):
                    smask = iota16 < pops[u]
                    pos = (csu + iota16) & (CAP - 1)
                    plsc.store_scatter(queue, [pos], svs[u], mask=smask)
                    csu = csu + pops[u]
                return csu

            cnt_splat = lax.fori_loop(0, C // (16 * FU), fbody, cnt_splat)
            cnt = cnt_splat[0]
            extra = jnp.where(c == N_CHUNKS - 1, SG - 1, 0)
            target = (cnt + extra) // SG
            nnew = target - nfired

            def bbody(j, _):
                qb = nfired + j

                @pl.when(lax.rem(qb, 2) == 0)
                def _():
                    stage_and_fire(qb, idx0, rows0, sem0)

                @pl.when(lax.rem(qb, 2) == 1)
                def _():
                    stage_and_fire(qb, idx1, rows1, sem1)

                @pl.when(qb >= 1)
                def _():
                    @pl.when(lax.rem(qb, 2) == 1)
                    def _():
                        drain_rows(rows0, sem0)
                        process_block(qb - 1, cnt_splat, rows0)

                    @pl.when(lax.rem(qb, 2) == 0)
                    def _():
                        drain_rows(rows1, sem1)
                        process_block(qb - 1, cnt_splat, rows1)

                return 0

            lax.fori_loop(0, nnew, bbody, 0)

            return (cnt_splat, target)

        cnt_splat, nfired = lax.fori_loop(
            0, N_CHUNKS, chunk_body, (jnp.zeros((16,), jnp.int32), jnp.int32(0))
        )

        @pl.when(nfired >= 1)
        def _():
            @pl.when(lax.rem(nfired, 2) == 1)
            def _():
                drain_rows(rows0, sem0)
                process_block(nfired - 1, cnt_splat, rows0)

            @pl.when(lax.rem(nfired, 2) == 0)
            def _():
                drain_rows(rows1, sem1)
                process_block(nfired - 1, cnt_splat, rows1)

        pltpu.sync_copy(
            acc.at[pl.ds(0, R * AS)], out_hbm.at[pl.ds(rid * R * AS, R * AS)]
        )
        return 0

    lax.fori_loop(0, 2, pass_body, 0)


def _sc_scatter_max(dst, h):
    mesh = plsc.VectorSubcoreMesh(core_axis_name="c", subcore_axis_name="s")
    f = functools.partial(
        pl.kernel,
        mesh=mesh,
        out_type=jax.ShapeDtypeStruct((N_NODES_P * AS,), jnp.float32),
        scratch_types=[
            pltpu.VMEM((2 * C,), jnp.int32),
            pltpu.VMEM((CAP,), jnp.int32),
            pltpu.VMEM((SG, DP), jnp.float32),
            pltpu.VMEM((SG, DP), jnp.float32),
            pltpu.VMEM((SG,), jnp.int32),
            pltpu.VMEM((SG,), jnp.int32),
            pltpu.VMEM(((R + 1) * AS,), jnp.float32),
            pltpu.SemaphoreType.DMA,
            pltpu.SemaphoreType.DMA,
            pltpu.SemaphoreType.DMA,
            pltpu.SemaphoreType.DMA,
        ],
        compiler_params=pltpu.CompilerParams(needs_layout_passes=False),
    )(_sc_body)
    return f(dst, h)


@jax.jit
def kernel(edge_index, src_emb, src_emb_in, W, b):
    del src_emb_in
    w_pad = jnp.zeros((D_IN, DP), jnp.float32).at[:, :D_IN].set(W.T)
    b_pad = jnp.zeros((1, DP), jnp.float32).at[0, :D_IN].set(b)
    h = _tc_h(src_emb, w_pad, b_pad)
    dst = edge_index[1]
    out = _sc_scatter_max(dst, h)
    return out.reshape(N_NODES_P, AS)[:N_NODES, :D_IN]


# speedup vs baseline: 1.8545x; 1.0064x over previous
import functools

import jax
import jax.numpy as jnp
from jax import lax
from jax.experimental import pallas as pl
from jax.experimental.pallas import tpu as pltpu
from jax.experimental.pallas import tpu_sc as plsc

N_NODES = 50000
N_EDGES = 800000
D_IN = 100
DP = 128
AS = 112
R = 784
N_RANGES = 64
N_NODES_P = N_RANGES * R
C = 4000
N_CHUNKS = N_EDGES // C
CAP = 8192
SG = 64
FU = 10
B_TC = 8000


def _mm_body(x_ref, w_ref, b_ref, o_ref):
    y = jnp.dot(x_ref[...], w_ref[...], preferred_element_type=jnp.float32)
    o_ref[...] = jnp.maximum(y + b_ref[...], 0.0)


def _tc_h(src_emb, w_pad, b_pad):
    return pl.pallas_call(
        _mm_body,
        grid=(N_EDGES // B_TC,),
        in_specs=[
            pl.BlockSpec((B_TC, D_IN), lambda i: (i, 0)),
            pl.BlockSpec((D_IN, DP), lambda i: (0, 0)),
            pl.BlockSpec((1, DP), lambda i: (0, 0)),
        ],
        out_specs=pl.BlockSpec((B_TC, DP), lambda i: (i, 0)),
        out_shape=jax.ShapeDtypeStruct((N_EDGES, DP), jnp.float32),
    )(src_emb, w_pad, b_pad)


def _splat_lane(v, r):
    idx = jnp.full((16,), r, jnp.int32)
    return lax.gather(
        v,
        idx[:, None],
        lax.GatherDimensionNumbers(
            offset_dims=(), collapsed_slice_dims=(0,), start_index_map=(0,)
        ),
        slice_sizes=(1,),
        mode=lax.GatherScatterMode.PROMISE_IN_BOUNDS,
    )


def _sc_body(dst_hbm, h_hbm, out_hbm, dstc, queue, rows0, rows1, idx0, idx1,
             acc, sem0, sem1, csem0, csem1):
    nc = 2
    wid = lax.axis_index("s") * nc + lax.axis_index("c")
    iota16 = lax.iota(jnp.int32, 16)
    neg_inf = jnp.full((16,), -jnp.inf, jnp.float32)

    def chunk_fire(c, half, csem):
        pltpu.make_async_copy(
            dst_hbm.at[pl.ds(c * C, C)], dstc.at[pl.ds(half * C, C)], csem
        ).start()

    def chunk_drain(half, csem):
        pltpu.make_async_copy(
            dst_hbm.at[pl.ds(0, C)], dstc.at[pl.ds(half * C, C)], csem
        ).wait()

    def stage_and_fire(qblk, idx_ref, rows_ref, sem):
        qoff = (qblk * SG) & (CAP - 1)
        for sub in range(SG // 16):
            qv = queue[pl.ds(qoff + sub * 16, 16)]
            ids = jnp.minimum(qv & 0xFFFFF, N_EDGES - 1)
            idx_ref[sub * 16:(sub + 1) * 16] = ids
        pltpu.make_async_copy(h_hbm.at[idx_ref], rows_ref, sem).start()

    def drain_rows(rows_ref, sem):
        pltpu.make_async_copy(h_hbm.at[pl.ds(0, SG)], rows_ref, sem).wait()

    def process_block(qblk, cnt_splat, rows_ref):
        qoff = (qblk * SG) & (CAP - 1)
        cnt = cnt_splat[0]
        for sub in range(SG // 16):
            qv = queue[pl.ds(qoff + sub * 16, 16)]
            d_raw = lax.shift_right_logical(qv, 20)
            gpos = qblk * SG + sub * 16
            for r in range(16):
                real_r = (gpos + r) < cnt
                d_s = jnp.where(real_r, d_raw[r], R)
                base = d_s * AS
                row = sub * 16 + r
                curs = []
                upds = []
                for k in range(AS // 16):
                    curs.append(acc[pl.ds(base + 16 * k, 16)])
                    upds.append(rows_ref[row, 16 * k:16 * (k + 1)])
                for k in range(AS // 16):
                    acc[pl.ds(base + 16 * k, 16)] = jnp.maximum(
                        curs[k], upds[k]
                    )

    def pass_body(p, _):
        rid = wid * 2 + p
        lo = rid * R

        def init_body(i, _):
            acc[pl.ds(i * 16, 16)] = neg_inf
            return 0

        lax.fori_loop(0, (R + 1) * AS // 16, init_body, 0)

        chunk_fire(0, 0, csem0)

        def chunk_body(c, carry):
            cnt_splat, nfired = carry
            half = lax.rem(c, 2)
            cbase = c * C

            @pl.when(half == 0)
            def _():
                chunk_drain(0, csem0)

            @pl.when(half == 1)
            def _():
                chunk_drain(1, csem1)

            @pl.when(c + 1 < N_CHUNKS)
            def _():
                @pl.when(half == 0)
                def _():
                    chunk_fire(c + 1, 1, csem1)

                @pl.when(half == 1)
                def _():
                    chunk_fire(c + 1, 0, csem0)

            dbase = half * C

            def fbody(i, cs):
                pops = []
                svs = []
                for u in range(FU):
                    off = i * 16 * FU + u * 16
                    v = dstc[pl.ds(dbase + off, 16)]
                    m = (v >= lo) & (v < lo + R)
                    packed = (lax.iota(jnp.int32, 16) + (cbase + off)) | (
                        (v - lo) << 20
                    )
                    _, sv = plsc.sort_key_val(
                        m.astype(jnp.int32), packed, descending=True
                    )
                    svs.append(sv)
                    pops.append(plsc.all_reduce_population_count(m))
                csu = cs
                for u in range(FU):
                    smask = iota16 < pops[u]
                    pos = (csu + iota16) & (CAP - 1)
                    plsc.store_scatter(queue, [pos], svs[u], mask=smask)
                    csu = csu + pops[u]
                return csu

            cnt_splat = lax.fori_loop(0, C // (16 * FU), fbody, cnt_splat)
            cnt = cnt_splat[0]
            extra = jnp.where(c == N_CHUNKS - 1, SG - 1, 0)
            target = (cnt + extra) // SG
            nnew = target - nfired

            def bbody(j, _):
                qb = nfired + j

                @pl.when(lax.rem(qb, 2) == 0)
                def _():
                    stage_and_fire(qb, idx0, rows0, sem0)

                @pl.when(lax.rem(qb, 2) == 1)
                def _():
                    stage_and_fire(qb, idx1, rows1, sem1)

                @pl.when(qb >= 1)
                def _():
                    @pl.when(lax.rem(qb, 2) == 1)
                    def _():
                        drain_rows(rows0, sem0)
                        process_block(qb - 1, cnt_splat, rows0)

                    @pl.when(lax.rem(qb, 2) == 0)
                    def _():
                        drain_rows(rows1, sem1)
                        process_block(qb - 1, cnt_splat, rows1)

                return 0

            lax.fori_loop(0, nnew, bbody, 0)

            return (cnt_splat, target)

        cnt_splat, nfired = lax.fori_loop(
            0, N_CHUNKS, chunk_body, (jnp.zeros((16,), jnp.int32), jnp.int32(0))
        )

        @pl.when(nfired >= 1)
        def _():
            @pl.when(lax.rem(nfired, 2) == 1)
            def _():
                drain_rows(rows0, sem0)
                process_block(nfired - 1, cnt_splat, rows0)

            @pl.when(lax.rem(nfired, 2) == 0)
            def _():
                drain_rows(rows1, sem1)
                process_block(nfired - 1, cnt_splat, rows1)

        pltpu.sync_copy(
            acc.at[pl.ds(0, R * AS)], out_hbm.at[pl.ds(rid * R * AS, R * AS)]
        )
        return 0

    lax.fori_loop(0, 2, pass_body, 0)


def _sc_scatter_max(dst, h):
    mesh = plsc.VectorSubcoreMesh(core_axis_name="c", subcore_axis_name="s")
    f = functools.partial(
        pl.kernel,
        mesh=mesh,
        out_type=jax.ShapeDtypeStruct((N_NODES_P * AS,), jnp.float32),
        scratch_types=[
            pltpu.VMEM((2 * C,), jnp.int32),
            pltpu.VMEM((CAP,), jnp.int32),
            pltpu.VMEM((SG, DP), jnp.float32),
            pltpu.VMEM((SG, DP), jnp.float32),
            pltpu.VMEM((SG,), jnp.int32),
            pltpu.VMEM((SG,), jnp.int32),
            pltpu.VMEM(((R + 1) * AS,), jnp.float32),
            pltpu.SemaphoreType.DMA,
            pltpu.SemaphoreType.DMA,
            pltpu.SemaphoreType.DMA,
            pltpu.SemaphoreType.DMA,
        ],
        compiler_params=pltpu.CompilerParams(needs_layout_passes=False),
    )(_sc_body)
    return f(dst, h)


@jax.jit
def kernel(edge_index, src_emb, src_emb_in, W, b):
    del src_emb_in
    w_pad = jnp.zeros((D_IN, DP), jnp.float32).at[:, :D_IN].set(W.T)
    b_pad = jnp.zeros((1, DP), jnp.float32).at[0, :D_IN].set(b)
    h = _tc_h(src_emb, w_pad, b_pad)
    dst = edge_index[1]
    out = _sc_scatter_max(dst, h)
    return out.reshape(N_NODES_P, AS)[:N_NODES, :D_IN]
